# baseline (speedup 1.0000x reference)
import numpy as _np

import jax
import jax.numpy as jnp
from jax import lax
from jax.experimental import pallas as pl
from jax.experimental.pallas import tpu as pltpu
from jax.experimental.pallas import tpu_sc as plsc

_N = 10000
_E = 320000
_D = 128
_L = 8
_EPS = 1e-5

_NC = 2
_NS = 16
_NW = _NC * _NS
_EA = _E + _N + 240
_EPT = _EA // _NW
_K = 40
_NCHUNK = _EPT // _K
_NBUF = 3
_NA = _N + 16
_RPS = _NA // _NS
_DW = 16

_PGRP = _np.concatenate([_np.arange(16) * 2, _np.arange(16) * 2 + 1])
_PERM = ((_np.arange(128) // 32) * 32 + _PGRP[_np.arange(128) % 32]).astype(_np.int32)


def _sc_body(mp_hbm, src_hbm, dst_hbm, out_hbm, src_v, dst_v,
             brows, frows, gsems, ssems, agg_sh):
    c = lax.axis_index("c")
    s = lax.axis_index("s")
    wid = c * _NS + s
    base = s * _RPS

    zero16 = jnp.zeros((16,), jnp.float32)

    def zbody(i, carry):
        frows[0][i // 8, pl.ds((i % 8) * 16, 16)] = zero16
        return carry

    lax.fori_loop(0, _K * (_D // 16), zbody, 0)
    nfull = _RPS // _K
    rem = _RPS - nfull * _K
    for t in range(nfull):
        pltpu.sync_copy(frows[0], agg_sh.at[pl.ds(base + t * _K, _K)])
    if rem:
        pltpu.sync_copy(frows[0].at[pl.ds(0, rem)],
                        agg_sh.at[pl.ds(base + nfull * _K, rem)])

    pltpu.sync_copy(src_hbm.at[wid], src_v)
    pltpu.sync_copy(dst_hbm.at[wid], dst_v)
    plsc.subcore_barrier()

    def gfire(j, b):
        pltpu.async_copy(mp_hbm.at[src_v.at[j]], brows[b], gsems[b])

    def gwait(j, b):
        pltpu.make_async_copy(mp_hbm.at[src_v.at[j]], brows[b], gsems[b]).wait()

    def sfire(j, b):
        pltpu.async_copy(frows[b], agg_sh.at[dst_v.at[j]], ssems[b], add=True)

    def swait(j, b):
        pltpu.make_async_copy(frows[b], agg_sh.at[dst_v.at[j]], ssems[b]).wait()

    himask = jnp.full((16,), -65536, jnp.int32)

    def convert(b):
        def crow(r8, carry):
            for dr in range(8):
                r = r8 * 8 + dr
                for g in range(4):
                    v = brows[b][r, pl.ds(32 * g, 32)]
                    vi = plsc.bitcast(v, jnp.int32)
                    lo = plsc.bitcast(vi << 16, jnp.float32)
                    hi = plsc.bitcast(vi & himask, jnp.float32)
                    frows[b][r, pl.ds(32 * g, 16)] = lo
                    frows[b][r, pl.ds(32 * g + 16, 16)] = hi
            return carry

        lax.fori_loop(0, _K // 8, crow, 0)

    for b in range(_NBUF):
        gfire(b, b)
    for b in range(_NBUF):
        gwait(b, b)
        convert(b)
        gfire(b + _NBUF, b)
        sfire(b, b)

    def loop_body(jq, carry):
        j = _NBUF * jq
        for b in range(_NBUF):
            idx = j + b
            gwait(idx, b)
            swait(idx - _NBUF, b)
            convert(b)

            @pl.when(idx + _NBUF < _NCHUNK)
            def _(idx=idx, b=b):
                gfire(idx + _NBUF, b)

            sfire(idx, b)
        return carry

    lax.fori_loop(1, _NCHUNK // _NBUF, loop_body, 0)
    for b in range(_NBUF):
        swait(_NCHUNK - _NBUF + b, b)
    plsc.subcore_barrier()
    pltpu.sync_copy(agg_sh.at[pl.ds(base, _RPS)], out_hbm.at[c].at[pl.ds(base, _RPS)])


_sc_call = pl.kernel(
    _sc_body,
    out_type=jax.ShapeDtypeStruct((_NC, _NA, _D), jnp.float32),
    mesh=plsc.VectorSubcoreMesh(core_axis_name="c", subcore_axis_name="s"),
    compiler_params=pltpu.CompilerParams(use_tc_tiling_on_sc=False,
                                         needs_layout_passes=False),
    scratch_types=[
        pltpu.VMEM((_NCHUNK, _K), jnp.int32),
        pltpu.VMEM((_NCHUNK, _K), jnp.int32),
        [pltpu.VMEM((_K, _D), jnp.bfloat16)] * _NBUF,
        [pltpu.VMEM((_K, _D), jnp.float32)] * _NBUF,
        [pltpu.SemaphoreType.DMA] * _NBUF,
        [pltpu.SemaphoreType.DMA] * _NBUF,
        pltpu.VMEM_SHARED((_NA, _D), jnp.float32),
    ],
)


def _deg_body(dst_hbm, out_hbm, dst_v, ones_v, zer_v, agg_sh, ssem0, ssem1):
    c = lax.axis_index("c")
    s = lax.axis_index("s")
    wid = c * _NS + s
    base = s * _RPS
    one16 = jnp.full((16,), 1.0, jnp.float32)
    zero16 = jnp.zeros((16,), jnp.float32)

    def fill(i, carry):
        ones_v[i, pl.ds(0, 16)] = one16
        zer_v[i, pl.ds(0, 16)] = zero16
        return carry

    lax.fori_loop(0, _K, fill, 0)

    nfull = _RPS // _K
    rem = _RPS - nfull * _K
    for t in range(nfull):
        pltpu.sync_copy(zer_v, agg_sh.at[pl.ds(base + t * _K, _K)])
    if rem:
        pltpu.sync_copy(zer_v.at[pl.ds(0, rem)],
                        agg_sh.at[pl.ds(base + nfull * _K, rem)])

    pltpu.sync_copy(dst_hbm.at[wid], dst_v)
    plsc.subcore_barrier()

    def sfire(j, sem):
        pltpu.async_copy(ones_v, agg_sh.at[dst_v.at[j]], sem, add=True)

    def swait(j, sem):
        pltpu.make_async_copy(ones_v, agg_sh.at[dst_v.at[j]], sem).wait()

    sfire(0, ssem0)
    sfire(1, ssem1)

    def body(j2, carry):
        j = 2 * j2
        swait(j, ssem0)

        @pl.when(j + 2 < _NCHUNK)
        def _():
            sfire(j + 2, ssem0)

        swait(j + 1, ssem1)

        @pl.when(j + 3 < _NCHUNK)
        def _():
            sfire(j + 3, ssem1)

        return carry

    lax.fori_loop(0, _NCHUNK // 2, body, 0)
    if _NCHUNK % 2:
        swait(_NCHUNK - 1, ssem0)
    plsc.subcore_barrier()
    pltpu.sync_copy(agg_sh.at[pl.ds(base, _RPS)], out_hbm.at[c].at[pl.ds(base, _RPS)])


_deg_call = pl.kernel(
    _deg_body,
    out_type=jax.ShapeDtypeStruct((_NC, _NA, _DW), jnp.float32),
    mesh=plsc.VectorSubcoreMesh(core_axis_name="c", subcore_axis_name="s"),
    compiler_params=pltpu.CompilerParams(use_tc_tiling_on_sc=False),
    scratch_types=[
        pltpu.VMEM((_NCHUNK, _K), jnp.int32),
        pltpu.VMEM((_K, _DW), jnp.float32),
        pltpu.VMEM((_K, _DW), jnp.float32),
        pltpu.VMEM_SHARED((_NA, _DW), jnp.float32),
        pltpu.SemaphoreType.DMA,
        pltpu.SemaphoreType.DMA,
    ],
)


_BR = 1000


def _prep_body(x_ref, wi_ref, bi_ref, d0_ref, d1_ref, h_ref, dinv_ref):
    h = jnp.dot(x_ref[...], wi_ref[...], preferred_element_type=jnp.float32)
    h_ref[...] = jnp.maximum(h + bi_ref[...], 0.0)
    deg = d0_ref[...] + d1_ref[...]
    dinv_ref[...] = jnp.broadcast_to(lax.rsqrt(deg)[:, :1], (_BR, _D))


def _mm_body(h_ref, w_ref, b_ref, scale_ref, out_ref):
    m = jnp.dot(h_ref[...], w_ref[...], preferred_element_type=jnp.float32)
    out_ref[...] = ((m + b_ref[...]) * scale_ref[...]).astype(jnp.bfloat16)


def _make_combine_body(scaled):
    def body(p0_ref, p1_ref, dinv_ref, res_ref, lns_ref, lnb_ref,
             w_ref, b_ref, h_ref, m_ref):
        pre = dinv_ref[...] * (p0_ref[...] + p1_ref[...])
        mu = jnp.mean(pre, axis=-1, keepdims=True)
        var = jnp.mean((pre - mu) ** 2, axis=-1, keepdims=True)
        xn = (pre - mu) / jnp.sqrt(var + _EPS) * lns_ref[...] + lnb_ref[...]
        h = jnp.maximum(xn, 0.0) + res_ref[...]
        h_ref[...] = h
        m = jnp.dot(h, w_ref[...], preferred_element_type=jnp.float32) + b_ref[...]
        if scaled:
            m_ref[...] = (m * dinv_ref[...]).astype(jnp.bfloat16)
        else:
            m_ref[...] = m

    return body


def _row_spec():
    return pl.BlockSpec((_BR, _D), lambda i: (i, 0))


def _deg_spec():
    return pl.BlockSpec((_BR, _DW), lambda i: (i, 0))


def _full_spec():
    return pl.BlockSpec((_D, _D), lambda i: (0, 0))


def _vec_spec():
    return pl.BlockSpec((1, _D), lambda i: (0, 0))


_GRID = (_N // _BR,)

_prep_call = pl.pallas_call(
    _prep_body,
    grid=_GRID,
    in_specs=[_row_spec(), _full_spec(), _vec_spec(), _deg_spec(), _deg_spec()],
    out_specs=[_row_spec(), _row_spec()],
    out_shape=[
        jax.ShapeDtypeStruct((_N, _D), jnp.float32),
        jax.ShapeDtypeStruct((_N, _D), jnp.float32),
    ],
)

_mm_call = pl.pallas_call(
    _mm_body,
    grid=_GRID,
    in_specs=[_row_spec(), _full_spec(), _vec_spec(), _row_spec()],
    out_specs=_row_spec(),
    out_shape=jax.ShapeDtypeStruct((_N, _D), jnp.bfloat16),
)


def _make_combine_call(scaled):
    return pl.pallas_call(
        _make_combine_body(scaled),
        grid=_GRID,
        in_specs=[_row_spec(), _row_spec(), _row_spec(), _row_spec(),
                  _vec_spec(), _vec_spec(), _full_spec(), _vec_spec()],
        out_specs=[_row_spec(), _row_spec()],
        out_shape=[
            jax.ShapeDtypeStruct((_N, _D), jnp.float32),
            jax.ShapeDtypeStruct((_N, _D),
                                 jnp.bfloat16 if scaled else jnp.float32),
        ],
    )


_combine_scaled = _make_combine_call(True)
_combine_plain = _make_combine_call(False)


def kernel(x, edge_index, Wi, bi, conv_W, conv_b, ln_s, ln_b, Wo, bo):
    perm = jnp.asarray(_PERM)
    loops = jnp.arange(_N, dtype=edge_index.dtype)
    npad = _EA - _E - _N
    src_all = jnp.concatenate(
        [edge_index[0], loops, jnp.zeros((npad,), edge_index.dtype)])
    dst_all = jnp.concatenate(
        [edge_index[1], loops, jnp.full((npad,), _N, edge_index.dtype)])
    src3 = src_all.reshape(_NW, _NCHUNK, _K)
    dst3 = dst_all.reshape(_NW, _NCHUNK, _K)

    Wi_p = Wi[:, perm]
    bi_p = bi[perm]
    ln_s_p = ln_s[:, perm]
    ln_b_p = ln_b[:, perm]
    conv_W_p = conv_W[:, perm, :]
    Wo_p = Wo[perm, :]

    degp = _deg_call(dst3)
    h, dinv = _prep_call(x, Wi_p, bi_p.reshape(1, _D),
                         degp[0], degp[1])
    mp = _mm_call(h, conv_W_p[0], conv_b[0].reshape(1, _D), dinv)

    for i in range(_L):
        parts = _sc_call(mp, src3, dst3)
        last = i + 1 == _L
        w_next = Wo_p if last else conv_W_p[i + 1]
        b_next = bo if last else conv_b[i + 1]
        combine = _combine_plain if last else _combine_scaled
        h, mp = combine(parts[0], parts[1], dinv, h,
                        ln_s_p[i].reshape(1, _D), ln_b_p[i].reshape(1, _D),
                        w_next, b_next.reshape(1, _D))
    return mp


# speedup vs baseline: 2.0179x; 2.0179x over previous
import jax
import jax.numpy as jnp
from jax import lax
from jax.experimental import pallas as pl
from jax.experimental.pallas import tpu as pltpu
from jax.experimental.pallas import tpu_sc as plsc

_N = 10000
_E = 320000
_D = 128
_L = 8
_EPS = 1e-5

_NC = 2
_NS = 16
_NW = _NC * _NS
_EPT = _E // _NW
_K = 40
_NCHUNK = _EPT // _K
_NBUF = 5
_RPS = _N // _NS
_DW = 16


def _sc_body(mp_hbm, src_hbm, dst_hbm, out_hbm, src_v, dst_v,
             rows, gsems, ssems, agg_sh):
    c = lax.axis_index("c")
    s = lax.axis_index("s")
    wid = c * _NS + s
    base = s * _RPS

    @pl.when(c == 0)
    def _():
        pltpu.sync_copy(mp_hbm.at[pl.ds(base, _RPS)], agg_sh.at[pl.ds(base, _RPS)])

    @pl.when(c != 0)
    def _():
        zero16 = jnp.zeros((16,), jnp.float32)

        def zbody(i, carry):
            rows[0][i // 8, pl.ds((i % 8) * 16, 16)] = zero16
            return carry

        lax.fori_loop(0, _K * (_D // 16), zbody, 0)
        nfull = _RPS // _K
        rem = _RPS - nfull * _K
        for t in range(nfull):
            pltpu.sync_copy(rows[0], agg_sh.at[pl.ds(base + t * _K, _K)])
        if rem:
            pltpu.sync_copy(rows[0].at[pl.ds(0, rem)],
                            agg_sh.at[pl.ds(base + nfull * _K, rem)])

    pltpu.sync_copy(src_hbm.at[wid], src_v)
    pltpu.sync_copy(dst_hbm.at[wid], dst_v)
    plsc.subcore_barrier()

    def gfire(j, b):
        pltpu.async_copy(mp_hbm.at[src_v.at[j]], rows[b], gsems[b])

    def gwait(j, b):
        pltpu.make_async_copy(mp_hbm.at[src_v.at[j]], rows[b], gsems[b]).wait()

    def sfire(j, b):
        pltpu.async_copy(rows[b], agg_sh.at[dst_v.at[j]], ssems[b], add=True)

    def swait(j, b):
        pltpu.make_async_copy(rows[b], agg_sh.at[dst_v.at[j]], ssems[b]).wait()

    for b in range(_NBUF):
        gfire(b, b)

    def body(jq, carry):
        j = _NBUF * jq
        for b in range(_NBUF):
            gwait(j + b, b)
            sfire(j + b, b)
            swait(j + b, b)

            @pl.when(j + b + _NBUF < _NCHUNK)
            def _(b=b):
                gfire(j + b + _NBUF, b)

        return carry

    nmain = _NCHUNK // _NBUF
    lax.fori_loop(0, nmain, body, 0)
    for r in range(nmain * _NBUF, _NCHUNK):
        b = r - nmain * _NBUF
        gwait(r, b)
        sfire(r, b)
        swait(r, b)
    plsc.subcore_barrier()
    pltpu.sync_copy(agg_sh.at[pl.ds(base, _RPS)], out_hbm.at[c].at[pl.ds(base, _RPS)])


_sc_call = pl.kernel(
    _sc_body,
    out_type=jax.ShapeDtypeStruct((_NC, _N, _D), jnp.float32),
    mesh=plsc.VectorSubcoreMesh(core_axis_name="c", subcore_axis_name="s"),
    compiler_params=pltpu.CompilerParams(use_tc_tiling_on_sc=False),
    scratch_types=[
        pltpu.VMEM((_NCHUNK, _K), jnp.int32),
        pltpu.VMEM((_NCHUNK, _K), jnp.int32),
        [pltpu.VMEM((_K, _D), jnp.float32)] * _NBUF,
        [pltpu.SemaphoreType.DMA] * _NBUF,
        [pltpu.SemaphoreType.DMA] * _NBUF,
        pltpu.VMEM_SHARED((_N, _D), jnp.float32),
    ],
)


def _deg_body(dst_hbm, out_hbm, dst_v, ones_v, zer_v, agg_sh, ssem0, ssem1):
    c = lax.axis_index("c")
    s = lax.axis_index("s")
    wid = c * _NS + s
    base = s * _RPS
    one16 = jnp.full((16,), 1.0, jnp.float32)
    zero16 = jnp.zeros((16,), jnp.float32)

    def fill(i, carry):
        ones_v[i, pl.ds(0, 16)] = one16
        zer_v[i, pl.ds(0, 16)] = zero16
        return carry

    lax.fori_loop(0, _K, fill, 0)

    nfull = _RPS // _K
    rem = _RPS - nfull * _K

    @pl.when(c == 0)
    def _():
        for t in range(nfull):
            pltpu.sync_copy(ones_v, agg_sh.at[pl.ds(base + t * _K, _K)])
        pltpu.sync_copy(ones_v.at[pl.ds(0, rem)],
                        agg_sh.at[pl.ds(base + nfull * _K, rem)])

    @pl.when(c != 0)
    def _():
        for t in range(nfull):
            pltpu.sync_copy(zer_v, agg_sh.at[pl.ds(base + t * _K, _K)])
        pltpu.sync_copy(zer_v.at[pl.ds(0, rem)],
                        agg_sh.at[pl.ds(base + nfull * _K, rem)])

    pltpu.sync_copy(dst_hbm.at[wid], dst_v)
    plsc.subcore_barrier()

    def sfire(j, sem):
        pltpu.async_copy(ones_v, agg_sh.at[dst_v.at[j]], sem, add=True)

    def swait(j, sem):
        pltpu.make_async_copy(ones_v, agg_sh.at[dst_v.at[j]], sem).wait()

    sfire(0, ssem0)
    sfire(1, ssem1)

    def body(j2, carry):
        j = 2 * j2
        swait(j, ssem0)

        @pl.when(j + 2 < _NCHUNK)
        def _():
            sfire(j + 2, ssem0)

        swait(j + 1, ssem1)

        @pl.when(j + 3 < _NCHUNK)
        def _():
            sfire(j + 3, ssem1)

        return carry

    lax.fori_loop(0, _NCHUNK // 2, body, 0)
    if _NCHUNK % 2:
        swait(_NCHUNK - 1, ssem0)
    plsc.subcore_barrier()
    pltpu.sync_copy(agg_sh.at[pl.ds(base, _RPS)], out_hbm.at[c].at[pl.ds(base, _RPS)])


_deg_call = pl.kernel(
    _deg_body,
    out_type=jax.ShapeDtypeStruct((_NC, _N, _DW), jnp.float32),
    mesh=plsc.VectorSubcoreMesh(core_axis_name="c", subcore_axis_name="s"),
    compiler_params=pltpu.CompilerParams(use_tc_tiling_on_sc=False),
    scratch_types=[
        pltpu.VMEM((_NCHUNK, _K), jnp.int32),
        pltpu.VMEM((_K, _DW), jnp.float32),
        pltpu.VMEM((_K, _DW), jnp.float32),
        pltpu.VMEM_SHARED((_N, _DW), jnp.float32),
        pltpu.SemaphoreType.DMA,
        pltpu.SemaphoreType.DMA,
    ],
)


_BR = 1000


def _prep_body(x_ref, wi_ref, bi_ref, d0_ref, d1_ref, h_ref, dinv_ref):
    h = jnp.dot(x_ref[...], wi_ref[...], preferred_element_type=jnp.float32)
    h_ref[...] = jnp.maximum(h + bi_ref[...], 0.0)
    deg = d0_ref[...] + d1_ref[...]
    dinv_ref[...] = jnp.broadcast_to(lax.rsqrt(deg)[:, :1], (_BR, _D))


def _mm_body(h_ref, w_ref, b_ref, scale_ref, out_ref):
    m = jnp.dot(h_ref[...], w_ref[...], preferred_element_type=jnp.float32)
    out_ref[...] = (m + b_ref[...]) * scale_ref[...]


def _make_combine_body(scaled):
    def body(p0_ref, p1_ref, dinv_ref, res_ref, lns_ref, lnb_ref,
             w_ref, b_ref, h_ref, m_ref):
        pre = dinv_ref[...] * (p0_ref[...] + p1_ref[...])
        mu = jnp.mean(pre, axis=-1, keepdims=True)
        var = jnp.mean((pre - mu) ** 2, axis=-1, keepdims=True)
        xn = (pre - mu) / jnp.sqrt(var + _EPS) * lns_ref[...] + lnb_ref[...]
        h = jnp.maximum(xn, 0.0) + res_ref[...]
        h_ref[...] = h
        m = jnp.dot(h, w_ref[...], preferred_element_type=jnp.float32) + b_ref[...]
        m_ref[...] = m * dinv_ref[...] if scaled else m

    return body


def _row_spec():
    return pl.BlockSpec((_BR, _D), lambda i: (i, 0))


def _deg_spec():
    return pl.BlockSpec((_BR, _DW), lambda i: (i, 0))


def _full_spec():
    return pl.BlockSpec((_D, _D), lambda i: (0, 0))


def _vec_spec():
    return pl.BlockSpec((1, _D), lambda i: (0, 0))


_GRID = (_N // _BR,)

_prep_call = pl.pallas_call(
    _prep_body,
    grid=_GRID,
    in_specs=[_row_spec(), _full_spec(), _vec_spec(), _deg_spec(), _deg_spec()],
    out_specs=[_row_spec(), _row_spec()],
    out_shape=[
        jax.ShapeDtypeStruct((_N, _D), jnp.float32),
        jax.ShapeDtypeStruct((_N, _D), jnp.float32),
    ],
)

_mm_call = pl.pallas_call(
    _mm_body,
    grid=_GRID,
    in_specs=[_row_spec(), _full_spec(), _vec_spec(), _row_spec()],
    out_specs=_row_spec(),
    out_shape=jax.ShapeDtypeStruct((_N, _D), jnp.float32),
)


def _make_combine_call(scaled):
    return pl.pallas_call(
        _make_combine_body(scaled),
        grid=_GRID,
        in_specs=[_row_spec(), _row_spec(), _row_spec(), _row_spec(),
                  _vec_spec(), _vec_spec(), _full_spec(), _vec_spec()],
        out_specs=[_row_spec(), _row_spec()],
        out_shape=[
            jax.ShapeDtypeStruct((_N, _D), jnp.float32),
            jax.ShapeDtypeStruct((_N, _D), jnp.float32),
        ],
    )


_combine_scaled = _make_combine_call(True)
_combine_plain = _make_combine_call(False)


def kernel(x, edge_index, Wi, bi, conv_W, conv_b, ln_s, ln_b, Wo, bo):
    src3 = edge_index[0].reshape(_NW, _NCHUNK, _K)
    dst3 = edge_index[1].reshape(_NW, _NCHUNK, _K)

    degp = _deg_call(dst3)
    h, dinv = _prep_call(x, Wi, bi.reshape(1, _D), degp[0], degp[1])
    mp = _mm_call(h, conv_W[0], conv_b[0].reshape(1, _D), dinv)

    for i in range(_L):
        parts = _sc_call(mp, src3, dst3)
        last = i + 1 == _L
        w_next = Wo if last else conv_W[i + 1]
        b_next = bo if last else conv_b[i + 1]
        combine = _combine_plain if last else _combine_scaled
        h, mp = combine(parts[0], parts[1], dinv, h,
                        ln_s[i].reshape(1, _D), ln_b[i].reshape(1, _D),
                        w_next, b_next.reshape(1, _D))
    return mp
